# baseline (device time: 13003 ns/iter reference)
import jax
import jax.numpy as jnp
from jax import lax
from jax.experimental import pallas as pl
from jax.experimental.pallas import tpu as pltpu

K = 8


def kernel(partial, resid, gamma):
    m, d = resid.shape
    rows = m // K
    gamma2d = gamma.reshape(1, d)

    def body(x_hbm, resid_hbm, gamma_hbm, out_hbm,
             xv, rv, gv, ov, send_buf, recv_buf,
             in_sems, aux_sems, out_sems, send_sems, recv_sems):
        my_x = lax.axis_index("x")
        my_y = lax.axis_index("y")
        my_z = lax.axis_index("z")
        partner = (1 - my_x, my_y, my_z)

        in_copies = []
        for k in range(K):
            sl = pl.ds(k * rows, rows)
            cp = pltpu.make_async_copy(
                x_hbm.at[0, sl], xv.at[sl], in_sems.at[k])
            cp.start()
            in_copies.append(cp)
        resid_cp = pltpu.make_async_copy(resid_hbm, rv, aux_sems.at[0])
        resid_cp.start()
        gamma_cp = pltpu.make_async_copy(gamma_hbm, gv, aux_sems.at[1])
        gamma_cp.start()

        barrier_sem = pltpu.get_barrier_semaphore()
        pl.semaphore_signal(
            barrier_sem, inc=1,
            device_id=partner, device_id_type=pl.DeviceIdType.MESH,
        )
        pl.semaphore_wait(barrier_sem, 1)

        rdmas = []
        for k in range(K):
            sl = pl.ds(k * rows, rows)
            in_copies[k].wait()
            q = jnp.clip(xv[sl] * 32.0, -127.0, 127.0)
            send_buf[sl] = jnp.round(q).astype(jnp.int8)
            rdma = pltpu.make_async_remote_copy(
                src_ref=send_buf.at[sl],
                dst_ref=recv_buf.at[sl],
                send_sem=send_sems.at[k],
                recv_sem=recv_sems.at[k],
                device_id=partner,
                device_id_type=pl.DeviceIdType.MESH,
            )
            rdma.start()
            rdmas.append(rdma)

        resid_cp.wait()
        gamma_cp.wait()

        out_copies = []
        for k in range(K):
            sl = pl.ds(k * rows, rows)
            rdmas[k].wait_recv()
            y = (xv[sl]
                 + recv_buf[sl].astype(jnp.float32) * (1.0 / 32.0)
                 + rv[sl])
            rms = jnp.sqrt(jnp.mean(y * y, axis=-1, keepdims=True) + 1e-6)
            ov[sl] = y / rms * gv[...]
            cp = pltpu.make_async_copy(
                ov.at[sl], out_hbm.at[sl], out_sems.at[k])
            cp.start()
            out_copies.append(cp)

        for k in range(K):
            out_copies[k].wait()
            rdmas[k].wait_send()

    return pl.pallas_call(
        body,
        out_shape=jax.ShapeDtypeStruct((m, d), jnp.float32),
        in_specs=[
            pl.BlockSpec(memory_space=pl.ANY),
            pl.BlockSpec(memory_space=pl.ANY),
            pl.BlockSpec(memory_space=pl.ANY),
        ],
        out_specs=pl.BlockSpec(memory_space=pl.ANY),
        scratch_shapes=[
            pltpu.VMEM((m, d), jnp.float32),
            pltpu.VMEM((m, d), jnp.float32),
            pltpu.VMEM((1, d), jnp.float32),
            pltpu.VMEM((m, d), jnp.float32),
            pltpu.VMEM((m, d), jnp.int8),
            pltpu.VMEM((m, d), jnp.int8),
            pltpu.SemaphoreType.DMA((K,)),
            pltpu.SemaphoreType.DMA((2,)),
            pltpu.SemaphoreType.DMA((K,)),
            pltpu.SemaphoreType.DMA((K,)),
            pltpu.SemaphoreType.DMA((K,)),
        ],
        compiler_params=pltpu.CompilerParams(collective_id=0),
    )(partial, resid, gamma2d)


# device time: 11503 ns/iter; 1.1304x vs baseline; 1.1304x over previous
import jax
import jax.numpy as jnp
from jax import lax
from jax.experimental import pallas as pl
from jax.experimental.pallas import tpu as pltpu

K = 2


def kernel(partial, resid, gamma):
    m, d = resid.shape
    rows = m // K
    gamma2d = gamma.reshape(1, d)

    def body(x_hbm, resid_hbm, gamma_hbm, out_hbm,
             xv, rv, gv, ov, send_buf, recv_buf,
             in_sems, aux_sems, out_sems, send_sems, recv_sems):
        my_x = lax.axis_index("x")
        my_y = lax.axis_index("y")
        my_z = lax.axis_index("z")
        partner = (1 - my_x, my_y, my_z)

        in_copies = []
        for k in range(K):
            sl = pl.ds(k * rows, rows)
            cp = pltpu.make_async_copy(
                x_hbm.at[0, sl], xv.at[sl], in_sems.at[k])
            cp.start()
            in_copies.append(cp)
        resid_cp = pltpu.make_async_copy(resid_hbm, rv, aux_sems.at[0])
        resid_cp.start()
        gamma_cp = pltpu.make_async_copy(gamma_hbm, gv, aux_sems.at[1])
        gamma_cp.start()

        barrier_sem = pltpu.get_barrier_semaphore()
        pl.semaphore_signal(
            barrier_sem, inc=1,
            device_id=partner, device_id_type=pl.DeviceIdType.MESH,
        )
        pl.semaphore_wait(barrier_sem, 1)

        rdmas = []
        for k in range(K):
            sl = pl.ds(k * rows, rows)
            in_copies[k].wait()
            q = jnp.clip(xv[sl] * 32.0, -127.0, 127.0)
            send_buf[sl] = jnp.round(q).astype(jnp.int8)
            rdma = pltpu.make_async_remote_copy(
                src_ref=send_buf.at[sl],
                dst_ref=recv_buf.at[sl],
                send_sem=send_sems.at[k],
                recv_sem=recv_sems.at[k],
                device_id=partner,
                device_id_type=pl.DeviceIdType.MESH,
            )
            rdma.start()
            rdmas.append(rdma)

        resid_cp.wait()
        gamma_cp.wait()

        out_copies = []
        for k in range(K):
            sl = pl.ds(k * rows, rows)
            rdmas[k].wait_recv()
            y = (xv[sl]
                 + recv_buf[sl].astype(jnp.float32) * (1.0 / 32.0)
                 + rv[sl])
            rms = jnp.sqrt(jnp.mean(y * y, axis=-1, keepdims=True) + 1e-6)
            ov[sl] = y / rms * gv[...]
            cp = pltpu.make_async_copy(
                ov.at[sl], out_hbm.at[sl], out_sems.at[k])
            cp.start()
            out_copies.append(cp)

        for k in range(K):
            out_copies[k].wait()
            rdmas[k].wait_send()

    return pl.pallas_call(
        body,
        out_shape=jax.ShapeDtypeStruct((m, d), jnp.float32),
        in_specs=[
            pl.BlockSpec(memory_space=pl.ANY),
            pl.BlockSpec(memory_space=pl.ANY),
            pl.BlockSpec(memory_space=pl.ANY),
        ],
        out_specs=pl.BlockSpec(memory_space=pl.ANY),
        scratch_shapes=[
            pltpu.VMEM((m, d), jnp.float32),
            pltpu.VMEM((m, d), jnp.float32),
            pltpu.VMEM((1, d), jnp.float32),
            pltpu.VMEM((m, d), jnp.float32),
            pltpu.VMEM((m, d), jnp.int8),
            pltpu.VMEM((m, d), jnp.int8),
            pltpu.SemaphoreType.DMA((K,)),
            pltpu.SemaphoreType.DMA((2,)),
            pltpu.SemaphoreType.DMA((K,)),
            pltpu.SemaphoreType.DMA((K,)),
            pltpu.SemaphoreType.DMA((K,)),
        ],
        compiler_params=pltpu.CompilerParams(collective_id=0),
    )(partial, resid, gamma2d)


# device time: 11443 ns/iter; 1.1363x vs baseline; 1.0052x over previous
import jax
import jax.numpy as jnp
from jax import lax
from jax.experimental import pallas as pl
from jax.experimental.pallas import tpu as pltpu

K = 4


def kernel(partial, resid, gamma):
    m, d = resid.shape
    rows = m // K
    gamma2d = gamma.reshape(1, d)

    def body(x_hbm, resid_hbm, gamma_hbm, out_hbm,
             xv, rv, gv, ov, send_buf, recv_buf,
             in_sems, aux_sems, out_sems, send_sems, recv_sems):
        my_x = lax.axis_index("x")
        my_y = lax.axis_index("y")
        my_z = lax.axis_index("z")
        partner = (1 - my_x, my_y, my_z)

        in_copies = []
        for k in range(K):
            sl = pl.ds(k * rows, rows)
            cp = pltpu.make_async_copy(
                x_hbm.at[0, sl], xv.at[sl], in_sems.at[k])
            cp.start()
            in_copies.append(cp)
        resid_cp = pltpu.make_async_copy(resid_hbm, rv, aux_sems.at[0])
        resid_cp.start()
        gamma_cp = pltpu.make_async_copy(gamma_hbm, gv, aux_sems.at[1])
        gamma_cp.start()

        barrier_sem = pltpu.get_barrier_semaphore()
        pl.semaphore_signal(barrier_sem, inc=1)
        pl.semaphore_wait(barrier_sem, 1)

        rdmas = []
        for k in range(K):
            sl = pl.ds(k * rows, rows)
            in_copies[k].wait()
            q = jnp.clip(xv[sl] * 32.0, -127.0, 127.0)
            send_buf[sl] = jnp.round(q).astype(jnp.int8)
            rdma = pltpu.make_async_remote_copy(
                src_ref=send_buf.at[sl],
                dst_ref=recv_buf.at[sl],
                send_sem=send_sems.at[k],
                recv_sem=recv_sems.at[k],
                device_id=partner,
                device_id_type=pl.DeviceIdType.MESH,
            )
            rdma.start()
            rdmas.append(rdma)

        resid_cp.wait()
        gamma_cp.wait()

        out_copies = []
        for k in range(K):
            sl = pl.ds(k * rows, rows)
            rdmas[k].wait_recv()
            y = (xv[sl]
                 + recv_buf[sl].astype(jnp.float32) * (1.0 / 32.0)
                 + rv[sl])
            rms = jnp.sqrt(jnp.mean(y * y, axis=-1, keepdims=True) + 1e-6)
            ov[sl] = y / rms * gv[...]
            cp = pltpu.make_async_copy(
                ov.at[sl], out_hbm.at[sl], out_sems.at[k])
            cp.start()
            out_copies.append(cp)

        for k in range(K):
            out_copies[k].wait()
            rdmas[k].wait_send()

    return pl.pallas_call(
        body,
        out_shape=jax.ShapeDtypeStruct((m, d), jnp.float32),
        in_specs=[
            pl.BlockSpec(memory_space=pl.ANY),
            pl.BlockSpec(memory_space=pl.ANY),
            pl.BlockSpec(memory_space=pl.ANY),
        ],
        out_specs=pl.BlockSpec(memory_space=pl.ANY),
        scratch_shapes=[
            pltpu.VMEM((m, d), jnp.float32),
            pltpu.VMEM((m, d), jnp.float32),
            pltpu.VMEM((1, d), jnp.float32),
            pltpu.VMEM((m, d), jnp.float32),
            pltpu.VMEM((m, d), jnp.int8),
            pltpu.VMEM((m, d), jnp.int8),
            pltpu.SemaphoreType.DMA((K,)),
            pltpu.SemaphoreType.DMA((2,)),
            pltpu.SemaphoreType.DMA((K,)),
            pltpu.SemaphoreType.DMA((K,)),
            pltpu.SemaphoreType.DMA((K,)),
        ],
        compiler_params=pltpu.CompilerParams(collective_id=0),
    )(partial, resid, gamma2d)
